# baseline (speedup 1.0000x reference)
import jax
import jax.numpy as jnp
from jax.experimental import pallas as pl
from jax.experimental.pallas import tpu as pltpu

_B = 32
_TS = 128
_G = 128
_M = 16
_T = _TS * _G * _M


def _pulse_body(xn_ref, on_ref, yn_ref, xs_ref, os_ref, o2_ref, st_ref,
                exclt_ref, e2_ref, wc_ref):
    m = pl.program_id(0)

    @pl.when(m == 0)
    def _init():
        e2_ref[...] = jnp.zeros_like(e2_ref)
        wc_ref[...] = jnp.zeros_like(wc_ref)

    xs_ref[...] = jnp.swapaxes(xn_ref[...], 1, 2)
    os_ref[...] = jnp.swapaxes(on_ref[...], 1, 2)

    def p1(i, run):
        return run + xs_ref[:, i, :]

    s = jax.lax.fori_loop(0, _TS, p1, jnp.zeros((_B, _G), jnp.float32),
                          unroll=8)

    st_ref[...] = jnp.swapaxes(s, 0, 1)
    e2 = e2_ref[0:1, :]

    def lvl2(n, run2):
        exclt_ref[pl.ds(n, 1), :] = run2 + e2
        return run2 + st_ref[pl.ds(n, 1), :]

    s2 = jax.lax.fori_loop(0, _G, lvl2, jnp.zeros((1, _B), jnp.float32),
                           unroll=8)
    e2_ref[0:1, :] = e2 + s2

    excl = jnp.swapaxes(exclt_ref[...], 0, 1)

    i127 = (s + excl) + os_ref[:, _TS - 1, :]
    w127 = i127 - jnp.floor(i127)
    wprev0 = jnp.concatenate([wc_ref[:, 0:1], w127[:, :-1]], axis=1)
    wc_ref[:, 0:1] = w127[:, _G - 1:_G]

    def p2(i, carry):
        run, wprev = carry
        x = xs_ref[:, i, :]
        run = run + x
        inst = run + excl + os_ref[:, i, :]
        w = inst - jnp.floor(inst)
        tr = (w - wprev) < 0
        val = jax.lax.rsqrt(jnp.where(tr, x, 1.0))
        o2_ref[:, i, :] = jnp.where(tr, val, 0.0)
        return run, w

    jax.lax.fori_loop(
        0, _TS, p2,
        (jnp.zeros((_B, _G), jnp.float32), wprev0), unroll=8)

    yn_ref[...] = jnp.swapaxes(o2_ref[...], 1, 2)


def kernel(phase, phase_offset):
    xn = phase.reshape(_B, _M * _G, _TS)
    on = phase_offset.reshape(_B, _M * _G, _TS)

    yn = pl.pallas_call(
        _pulse_body,
        grid=(_M,),
        in_specs=[
            pl.BlockSpec((_B, _G, _TS), lambda m: (0, m, 0)),
            pl.BlockSpec((_B, _G, _TS), lambda m: (0, m, 0)),
        ],
        out_specs=pl.BlockSpec((_B, _G, _TS), lambda m: (0, m, 0)),
        out_shape=jax.ShapeDtypeStruct((_B, _M * _G, _TS), jnp.float32),
        scratch_shapes=[
            pltpu.VMEM((_B, _TS, _G), jnp.float32),
            pltpu.VMEM((_B, _TS, _G), jnp.float32),
            pltpu.VMEM((_B, _TS, _G), jnp.float32),
            pltpu.VMEM((_G, _B), jnp.float32),
            pltpu.VMEM((_G, _B), jnp.float32),
            pltpu.VMEM((8, _B), jnp.float32),
            pltpu.VMEM((_B, 128), jnp.float32),
        ],
    )(xn, on)

    return yn.reshape(_B, _T)


# speedup vs baseline: 1.5671x; 1.5671x over previous
import jax
import jax.numpy as jnp
from jax.experimental import pallas as pl
from jax.experimental.pallas import tpu as pltpu

_B = 32
_TS = 128
_G = 128
_M = 16
_T = _TS * _G * _M


def _pulse_body(xn_ref, on_ref, yn_ref, xs_ref, os_ref, o2_ref, st_ref,
                exclt_ref, e2_ref, wc_ref):
    m = pl.program_id(0)

    @pl.when(m == 0)
    def _init():
        e2_ref[...] = jnp.zeros_like(e2_ref)
        wc_ref[...] = jnp.zeros_like(wc_ref)

    xs_ref[...] = jnp.swapaxes(xn_ref[...].reshape(_B, _G, _TS), 1, 2)
    os_ref[...] = jnp.swapaxes(on_ref[...].reshape(_B, _G, _TS), 1, 2)

    def p1(i, run):
        return run + xs_ref[:, i, :]

    s = jax.lax.fori_loop(0, _TS, p1, jnp.zeros((_B, _G), jnp.float32),
                          unroll=8)

    st_ref[...] = jnp.swapaxes(s, 0, 1)
    e2 = e2_ref[0:1, :]

    def lvl2(n, run2):
        exclt_ref[pl.ds(n, 1), :] = run2 + e2
        return run2 + st_ref[pl.ds(n, 1), :]

    s2 = jax.lax.fori_loop(0, _G, lvl2, jnp.zeros((1, _B), jnp.float32),
                           unroll=8)
    e2_ref[0:1, :] = e2 + s2

    excl = jnp.swapaxes(exclt_ref[...], 0, 1)

    i127 = (s + excl) + os_ref[:, _TS - 1, :]
    w127 = i127 - jnp.floor(i127)
    wprev0 = jnp.concatenate([wc_ref[:, 0:1], w127[:, :-1]], axis=1)
    wc_ref[:, 0:1] = w127[:, _G - 1:_G]

    def p2(i, carry):
        run, wprev = carry
        x = xs_ref[:, i, :]
        run = run + x
        inst = run + excl + os_ref[:, i, :]
        w = inst - jnp.floor(inst)
        tr = (w - wprev) < 0
        val = jax.lax.rsqrt(jnp.where(tr, x, 1.0))
        o2_ref[:, i, :] = jnp.where(tr, val, 0.0)
        return run, w

    jax.lax.fori_loop(
        0, _TS, p2,
        (jnp.zeros((_B, _G), jnp.float32), wprev0), unroll=8)

    yn_ref[...] = jnp.swapaxes(o2_ref[...], 1, 2).reshape(_B, _G * _TS)


def kernel(phase, phase_offset):
    _C = _G * _TS
    yn = pl.pallas_call(
        _pulse_body,
        grid=(_M,),
        in_specs=[
            pl.BlockSpec((_B, _C), lambda m: (0, m)),
            pl.BlockSpec((_B, _C), lambda m: (0, m)),
        ],
        out_specs=pl.BlockSpec((_B, _C), lambda m: (0, m)),
        out_shape=jax.ShapeDtypeStruct((_B, _T), jnp.float32),
        scratch_shapes=[
            pltpu.VMEM((_B, _TS, _G), jnp.float32),
            pltpu.VMEM((_B, _TS, _G), jnp.float32),
            pltpu.VMEM((_B, _TS, _G), jnp.float32),
            pltpu.VMEM((_G, _B), jnp.float32),
            pltpu.VMEM((_G, _B), jnp.float32),
            pltpu.VMEM((8, _B), jnp.float32),
            pltpu.VMEM((_B, 128), jnp.float32),
        ],
    )(phase, phase_offset)

    return yn


# speedup vs baseline: 2.4136x; 1.5401x over previous
import jax
import jax.numpy as jnp
from jax.experimental import pallas as pl
from jax.experimental.pallas import tpu as pltpu

_B = 32
_TS = 128
_G = 128
_M = 16
_T = _TS * _G * _M


def _pulse_body(xn_ref, on_ref, yn_ref, xs_ref, inner_ref, exclt_ref, e2_ref,
                wc_ref):
    m = pl.program_id(0)

    @pl.when(m == 0)
    def _init():
        e2_ref[...] = jnp.zeros_like(e2_ref)
        wc_ref[...] = jnp.zeros_like(wc_ref)

    xn = xn_ref[...]

    xs_ref[...] = jnp.transpose(xn.reshape(_B, _G, _TS), (2, 0, 1))

    def p1(i, run):
        run = run + xs_ref[i]
        inner_ref[i] = run
        return run

    s = jax.lax.fori_loop(0, _TS, p1, jnp.zeros((_B, _G), jnp.float32),
                          unroll=8)

    st = jnp.swapaxes(s, 0, 1)
    e2 = e2_ref[0:1, :]
    run2 = jnp.zeros((1, _B), jnp.float32)
    for n in range(_G):
        exclt_ref[n:n + 1, :] = run2 + e2
        run2 = run2 + st[n:n + 1, :]
    e2_ref[0:1, :] = e2 + run2

    excl = jnp.swapaxes(exclt_ref[...], 0, 1)

    cum_i = inner_ref[...] + excl[None, :, :]
    cum = jnp.transpose(cum_i, (1, 2, 0)).reshape(_B, _G * _TS)

    inst = cum + on_ref[...]
    w = inst - jnp.floor(inst)
    wprev = jnp.concatenate([wc_ref[:, 0:1], w[:, :-1]], axis=1)
    wc_ref[:, 0:1] = w[:, _G * _TS - 1:]
    tr = (w - wprev) < 0
    val = jax.lax.rsqrt(jnp.where(tr, xn, 1.0))
    yn_ref[...] = jnp.where(tr, val, 0.0)


def kernel(phase, phase_offset):
    _C = _G * _TS
    yn = pl.pallas_call(
        _pulse_body,
        grid=(_M,),
        in_specs=[
            pl.BlockSpec((_B, _C), lambda m: (0, m)),
            pl.BlockSpec((_B, _C), lambda m: (0, m)),
        ],
        out_specs=pl.BlockSpec((_B, _C), lambda m: (0, m)),
        out_shape=jax.ShapeDtypeStruct((_B, _T), jnp.float32),
        scratch_shapes=[
            pltpu.VMEM((_TS, _B, _G), jnp.float32),
            pltpu.VMEM((_TS, _B, _G), jnp.float32),
            pltpu.VMEM((_G, _B), jnp.float32),
            pltpu.VMEM((8, _B), jnp.float32),
            pltpu.VMEM((_B, 128), jnp.float32),
        ],
    )(phase, phase_offset)

    return yn
